# baseline (speedup 1.0000x reference)
import functools

import jax
import jax.numpy as jnp
import numpy as np
from jax import lax
from jax.experimental import pallas as pl
from jax.experimental.pallas import tpu as pltpu
from jax.experimental.pallas import tpu_sc as plsc

H = 48
W = 48
STRIDE = 16
A = 9
C_IN = 256
N_ANCH = H * W * A
ROWS = N_ANCH // 128
C = 6144
CROWS = C // 128
CPAD = C + 128
PRE_NMS = 6000
POST_NMS = 1000
NMS_THRESH = 0.7
MIN_SIZE = 1e-3
BBOX_CLIP = float(np.log(1000.0 / 16.0))
NEG_INF = float("-inf")

N_SUB = 16
SLOTS_PER_W = C // N_SUB


def _conv_heads_kernel(x_ref, w9_ref, bconv_ref, wh_ref, bh_ref, out_ref):
    acc = jnp.zeros((H * W, C_IN), dtype=jnp.float32)
    for j in range(9):
        dh, dw = j // 3, j % 3
        xs = x_ref[dh:dh + H, dw:dw + W, :].reshape(H * W, C_IN)
        acc = acc + jnp.dot(xs, w9_ref[j], preferred_element_type=jnp.float32)
    t = jnp.maximum(acc + bconv_ref[...], 0.0)
    out_ref[...] = jnp.dot(t, wh_ref[...],
                           preferred_element_type=jnp.float32) + bh_ref[...]


def _decode_rank_kernel(s_ref, dx_ref, dy_ref, dw_ref, dh_ref,
                        aw_ref, ah_ref, acx_ref, acy_ref,
                        x1_ref, y1_ref, x2_ref, y2_ref, sc_ref, dst_ref,
                        *, img_w, img_h):
    S = s_ref[...]
    aw = aw_ref[...]
    ah = ah_ref[...]
    dwc = jnp.minimum(dw_ref[...], BBOX_CLIP)
    dhc = jnp.minimum(dh_ref[...], BBOX_CLIP)
    pcx = dx_ref[...] * aw + acx_ref[...]
    pcy = dy_ref[...] * ah + acy_ref[...]
    pw = jnp.exp(dwc) * aw
    ph = jnp.exp(dhc) * ah
    X1 = jnp.clip(pcx - 0.5 * pw, 0.0, img_w)
    Y1 = jnp.clip(pcy - 0.5 * ph, 0.0, img_h)
    X2 = jnp.clip(pcx + 0.5 * pw, 0.0, img_w)
    Y2 = jnp.clip(pcy + 0.5 * ph, 0.0, img_h)
    valid = jnp.logical_and(X2 - X1 >= MIN_SIZE, Y2 - Y1 >= MIN_SIZE)

    b = jax.lax.bitcast_convert_type(S, jnp.int32)
    key = b ^ ((b >> 31) & jnp.int32(0x7FFFFFFF))
    kmin = jnp.min(key) - 1
    kmax = jnp.max(key)

    def _bs_val(_, st):
        lo, hi = st
        mid = (lo & hi) + ((lo ^ hi) >> 1)
        g = jnp.sum(jnp.where(key > mid, 1, 0).astype(jnp.int32))
        take_lo = g >= PRE_NMS
        return (jnp.where(take_lo, mid, lo), jnp.where(take_lo, hi, mid))

    _, thr = jax.lax.fori_loop(0, 33, _bs_val, (kmin, kmax))
    g_cnt = jnp.sum(jnp.where(key > thr, 1, 0).astype(jnp.int32))
    eq = key == thr

    rows_i = jax.lax.broadcasted_iota(jnp.int32, (ROWS, 128), 0)
    cols_i = jax.lax.broadcasted_iota(jnp.int32, (ROWS, 128), 1)
    iota = rows_i * 128 + cols_i

    def _bs_idx(_, st):
        lo, hi = st
        mid = (lo + hi) // 2
        cnt = g_cnt + jnp.sum(
            jnp.where(jnp.logical_and(eq, iota <= mid), 1, 0).astype(jnp.int32))
        ok = cnt >= PRE_NMS
        return (jnp.where(ok, lo, mid), jnp.where(ok, mid, hi))

    _, cutoff = jax.lax.fori_loop(0, 16, _bs_idx,
                                  (jnp.int32(-1), jnp.int32(N_ANCH - 1)))
    sel = jnp.logical_or(key > thr, jnp.logical_and(eq, iota <= cutoff))
    selv = jnp.where(jnp.logical_and(sel, valid), 1.0, 0.0).astype(jnp.float32)

    u_rows = jax.lax.broadcasted_iota(jnp.int32, (128, 128), 0)
    u_cols = jax.lax.broadcasted_iota(jnp.int32, (128, 128), 1)
    U = jnp.where(u_rows <= u_cols, 1.0, 0.0).astype(jnp.float32)
    lane_pref = jnp.dot(selv, U, preferred_element_type=jnp.float32)
    rowsum = lane_pref[:, 127:128]
    m_rows = jax.lax.broadcasted_iota(jnp.int32, (ROWS, ROWS), 0)
    m_cols = jax.lax.broadcasted_iota(jnp.int32, (ROWS, ROWS), 1)
    Mstrict = jnp.where(m_cols < m_rows, 1.0, 0.0).astype(jnp.float32)
    rowexcl = jnp.dot(Mstrict, rowsum, preferred_element_type=jnp.float32)
    rank = rowexcl + lane_pref - 1.0
    dst = jnp.where(selv > 0.0, rank.astype(jnp.int32), jnp.int32(C))

    x1_ref[...] = X1
    y1_ref[...] = Y1
    x2_ref[...] = X2
    y2_ref[...] = Y2
    sc_ref[...] = jax.nn.sigmoid(S)
    dst_ref[...] = dst


def _sc_compact_kernel(dst_hbm, table_hbm, rows_hbm, flag_hbm, srcidx_hbm,
                       dstv, locv, sidxv, flagv, idxs, rowsv, sem):
    c_id = lax.axis_index("c")
    s_id = lax.axis_index("s")

    @pl.when(jnp.logical_and(c_id == 0, s_id == 0))
    def _scatter():
        pltpu.sync_copy(dst_hbm, dstv)
        zz = jnp.zeros((16,), jnp.int32)

        def zb(i, carry):
            locv[pl.ds(i * 16, 16)] = zz
            return carry

        lax.fori_loop(0, CPAD // 16, zb, 0)
        ii = lax.iota(jnp.int32, 16)

        def sb(i, carry):
            d = dstv[pl.ds(i * 16, 16)]
            sel16 = d < C
            d2 = jnp.where(sel16, d, C + ii)
            src = ii + i * 16 + 1
            plsc.store_scatter(locv, [d2], src, mask=sel16)
            return carry

        lax.fori_loop(0, N_ANCH // 16, sb, 0)

        def rb(i, acc):
            return acc + jnp.sum(locv[pl.ds(i * 16, 16)])

        chk = lax.fori_loop(0, CPAD // 16, rb, jnp.int32(0))
        locv[pl.ds(C + 16, 16)] = jnp.zeros((16,), jnp.int32) + chk * 0
        pltpu.sync_copy(locv, srcidx_hbm)

    plsc.subcore_barrier()

    @pl.when(c_id == 0)
    def _gather():
        base = s_id * SLOTS_PER_W
        pltpu.sync_copy(srcidx_hbm.at[pl.ds(base, SLOTS_PER_W)], sidxv)

        def gb(k, carry):
            s16 = sidxv[pl.ds(k * 16, 16)]
            flagv[pl.ds(k * 16, 16)] = jnp.where(s16 != 0, 1, 0).astype(jnp.int32)
            idxs[k // 8, pl.ds((k % 8) * 16, 16)] = jnp.maximum(s16 - 1, 0)
            return carry

        lax.fori_loop(0, SLOTS_PER_W // 16, gb, 0)

        def rb2(k, acc):
            return (acc + jnp.sum(idxs[k // 8, pl.ds((k % 8) * 16, 16)])
                    + jnp.sum(flagv[pl.ds(k * 16, 16)]))

        chk2 = lax.fori_loop(0, SLOTS_PER_W // 16, rb2, jnp.int32(0))
        flagv[pl.ds(0, 16)] = flagv[pl.ds(0, 16)] + chk2 * 0
        pltpu.sync_copy(flagv, flag_hbm.at[pl.ds(base, SLOTS_PER_W)])
        for ch in range(SLOTS_PER_W // 128):
            pltpu.async_copy(table_hbm.at[idxs.at[ch]], rowsv, sem).wait()
            pltpu.sync_copy(rowsv, rows_hbm.at[pl.ds(base + ch * 128, 128)])


def _nms_kernel(s_ref, f_ref, x1_ref, y1_ref, x2_ref, y2_ref,
                ox1_ref, oy1_ref, ox2_ref, oy2_ref, osc_ref,
                sx1_ref, sy1_ref, sx2_ref, sy2_ref):
    X1 = x1_ref[...]
    Y1 = y1_ref[...]
    X2 = x2_ref[...]
    Y2 = y2_ref[...]
    areas = (X2 - X1) * (Y2 - Y1)
    neg = jnp.float32(NEG_INF)
    s0 = jnp.where(f_ref[...] != 0, s_ref[...], neg)
    sx1_ref[...] = X1
    sy1_ref[...] = Y1
    sx2_ref[...] = X2
    sy2_ref[...] = Y2
    rows_i = jax.lax.broadcasted_iota(jnp.int32, (CROWS, 128), 0)
    cols_i = jax.lax.broadcasted_iota(jnp.int32, (CROWS, 128), 1)
    iota = rows_i * 128 + cols_i
    lane = jax.lax.broadcasted_iota(jnp.int32, (1, 128), 1)
    big = jnp.int32(1 << 30)

    def _nms_body(i, s):
        m = jnp.max(s)
        validm = m > neg
        idx = jnp.min(jnp.where(s == m, iota, big))
        r = idx // 128
        oh = (lane == (idx - r * 128)).astype(jnp.float32)
        bx1 = jnp.sum(sx1_ref[pl.ds(r, 1), :] * oh)
        by1 = jnp.sum(sy1_ref[pl.ds(r, 1), :] * oh)
        bx2 = jnp.sum(sx2_ref[pl.ds(r, 1), :] * oh)
        by2 = jnp.sum(sy2_ref[pl.ds(r, 1), :] * oh)
        a1 = (bx2 - bx1) * (by2 - by1)
        iw = jnp.maximum(jnp.minimum(bx2, X2) - jnp.maximum(bx1, X1), 0.0)
        ih = jnp.maximum(jnp.minimum(by2, Y2) - jnp.maximum(by1, Y1), 0.0)
        inter = iw * ih
        s2 = jnp.where(inter > NMS_THRESH * (a1 + areas - inter + 1e-9), neg, s)
        zrow = jnp.zeros((1, 128), dtype=jnp.float32)
        fv = jnp.where(validm, 1.0, 0.0)
        ox1_ref[pl.ds(i, 1), :] = zrow + bx1 * fv
        oy1_ref[pl.ds(i, 1), :] = zrow + by1 * fv
        ox2_ref[pl.ds(i, 1), :] = zrow + bx2 * fv
        oy2_ref[pl.ds(i, 1), :] = zrow + by2 * fv
        osc_ref[pl.ds(i, 1), :] = zrow + jnp.where(validm, m, 0.0)
        return s2

    jax.lax.fori_loop(0, POST_NMS, _nms_body, s0)


def _anchor_planes():
    sizes = np.array([32.0, 64.0, 128.0], dtype=np.float32)
    ratios = np.array([0.5, 1.0, 2.0], dtype=np.float32)
    h_r = np.sqrt(ratios)
    w_r = 1.0 / h_r
    ws = (w_r[:, None] * sizes[None, :]).reshape(-1)
    hs = (h_r[:, None] * sizes[None, :]).reshape(-1)
    cell = np.round(np.stack([-ws, -hs, ws, hs], axis=1) / 2.0).astype(np.float32)
    sx = np.arange(W, dtype=np.float32) * STRIDE
    sy = np.arange(H, dtype=np.float32) * STRIDE
    gy, gx = np.meshgrid(sy, sx, indexing="ij")
    shifts = np.stack([gx.reshape(-1), gy.reshape(-1),
                       gx.reshape(-1), gy.reshape(-1)], axis=1)
    anch = (shifts[:, None, :] + cell[None, :, :]).reshape(-1, 4)
    widths = anch[:, 2] - anch[:, 0]
    heights = anch[:, 3] - anch[:, 1]
    ctr_x = anch[:, 0] + 0.5 * widths
    ctr_y = anch[:, 1] + 0.5 * heights
    shape = (ROWS, 128)
    return (jnp.asarray(widths.reshape(shape)),
            jnp.asarray(heights.reshape(shape)),
            jnp.asarray(ctr_x.reshape(shape)),
            jnp.asarray(ctr_y.reshape(shape)))


def kernel(images, features, w_conv, b_conv, w_obj, b_obj, w_box, b_box):
    img_h = float(images.shape[2])
    img_w = float(images.shape[3])

    x = jnp.transpose(features[0], (1, 2, 0))
    xpad = jnp.pad(x, ((1, 1), (1, 1), (0, 0)))
    w9 = jnp.transpose(w_conv, (2, 3, 1, 0)).reshape(9, C_IN, C_IN)
    wobj = jnp.transpose(w_obj[:, :, 0, 0], (1, 0))
    wbox = jnp.transpose(w_box[:, :, 0, 0], (1, 0))
    wh = jnp.concatenate(
        [wobj, wbox, jnp.zeros((C_IN, 128 - A - 4 * A), jnp.float32)], axis=1)
    bh = jnp.concatenate(
        [b_obj, b_box, jnp.zeros((128 - A - 4 * A,), jnp.float32)])[None, :]

    heads = pl.pallas_call(
        _conv_heads_kernel,
        out_shape=jax.ShapeDtypeStruct((H * W, 128), jnp.float32),
    )(xpad, w9, b_conv[None, :], wh, bh)

    obj = heads[:, :A].reshape(ROWS, 128)
    deltas = heads[:, A:A + 4 * A].reshape(H * W, A, 4)
    dxp = deltas[:, :, 0].reshape(ROWS, 128)
    dyp = deltas[:, :, 1].reshape(ROWS, 128)
    dwp = deltas[:, :, 2].reshape(ROWS, 128)
    dhp = deltas[:, :, 3].reshape(ROWS, 128)
    aw, ah, acx, acy = _anchor_planes()

    plane = jax.ShapeDtypeStruct((ROWS, 128), jnp.float32)
    X1, Y1, X2, Y2, SC, DST = pl.pallas_call(
        functools.partial(_decode_rank_kernel, img_w=img_w, img_h=img_h),
        out_shape=[plane, plane, plane, plane, plane,
                   jax.ShapeDtypeStruct((ROWS, 128), jnp.int32)],
    )(obj, dxp, dyp, dwp, dhp, aw, ah, acx, acy)

    table = jnp.stack([X1.reshape(-1), Y1.reshape(-1), X2.reshape(-1),
                       Y2.reshape(-1), SC.reshape(-1)], axis=1)
    table = jnp.concatenate(
        [table, jnp.zeros((N_ANCH, 123), jnp.float32)], axis=1)

    mesh = plsc.VectorSubcoreMesh(core_axis_name="c", subcore_axis_name="s")
    sc_fn = functools.partial(
        pl.kernel,
        mesh=mesh,
        out_type=[jax.ShapeDtypeStruct((C, 128), jnp.float32),
                  jax.ShapeDtypeStruct((C,), jnp.int32),
                  jax.ShapeDtypeStruct((CPAD,), jnp.int32)],
        scratch_types=[pltpu.VMEM((N_ANCH,), jnp.int32),
                       pltpu.VMEM((CPAD,), jnp.int32),
                       pltpu.VMEM((SLOTS_PER_W,), jnp.int32),
                       pltpu.VMEM((SLOTS_PER_W,), jnp.int32),
                       pltpu.VMEM((SLOTS_PER_W // 128, 128), jnp.int32),
                       pltpu.VMEM((128, 128), jnp.float32),
                       pltpu.SemaphoreType.DMA],
        compiler_params=pltpu.CompilerParams(needs_layout_passes=False),
    )(_sc_compact_kernel)
    rows, flag, _ = sc_fn(DST.reshape(-1), table)

    scp = rows[:, 4].reshape(CROWS, 128)
    fp = flag.reshape(CROWS, 128)
    x1c = rows[:, 0].reshape(CROWS, 128)
    y1c = rows[:, 1].reshape(CROWS, 128)
    x2c = rows[:, 2].reshape(CROWS, 128)
    y2c = rows[:, 3].reshape(CROWS, 128)

    out_shapes = [jax.ShapeDtypeStruct((POST_NMS, 128), jnp.float32)] * 5
    ox1, oy1, ox2, oy2, osc = pl.pallas_call(
        _nms_kernel,
        out_shape=out_shapes,
        scratch_shapes=[pltpu.VMEM((CROWS, 128), jnp.float32)] * 4,
    )(scp, fp, x1c, y1c, x2c, y2c)

    out_boxes = jnp.stack(
        [ox1[:, 0], oy1[:, 0], ox2[:, 0], oy2[:, 0]], axis=1)
    out_scores = osc[:, 0]
    return out_boxes, out_scores
